# Initial kernel scaffold; baseline (speedup 1.0000x reference)
import jax
import jax.numpy as jnp
from jax.experimental import pallas as pl


def kernel(inputs):
    raise NotImplementedError("write your pallas kernel here")


# speedup vs baseline: 9.3988x; 9.3988x over previous
import functools
from functools import partial

import jax
import jax.numpy as jnp
from jax.experimental import pallas as pl
from jax.experimental.pallas import tpu as pltpu

_MASK_THRESHOLD = 1e8
_TARGET_RATE = 0.5


def _thresh_body(x_ref, t_ref, bits_ref):
    x = x_ref[...]
    bits_ref[...] = jax.lax.bitcast_convert_type(jnp.abs(x), jnp.int32)
    wn = jnp.sum((x < _MASK_THRESHOLD).astype(jnp.float32), axis=0)
    rank = (_TARGET_RATE * wn).astype(jnp.int32) - 1

    def round_fn(j, prefix):
        bit = jnp.int32(1) << (30 - j)
        cand = prefix | bit
        cnt = jnp.sum((bits_ref[...] < cand[None, :]).astype(jnp.int32), axis=0)
        return jnp.where(cnt <= rank, cand, prefix)

    prefix = jax.lax.fori_loop(0, 31, round_fn, jnp.zeros(rank.shape, jnp.int32))
    t_ref[...] = jax.lax.bitcast_convert_type(prefix, jnp.float32)


def _mask_body(x_ref, t_ref, o_ref):
    x = x_ref[...]
    t = t_ref[...]
    o_ref[...] = jnp.where(jnp.abs(x) <= t[None, :], jnp.float32(0.0), x)


@jax.jit
def kernel(inputs):
    shape = inputs.shape
    C = shape[-1]
    N = 1
    for s in shape[:-1]:
        N *= s
    x2 = inputs.reshape(N, C)

    ch_tile = min(128, C)
    thresh = pl.pallas_call(
        _thresh_body,
        grid=(C // ch_tile,),
        in_specs=[pl.BlockSpec((N, ch_tile), lambda i: (0, i))],
        out_specs=pl.BlockSpec((ch_tile,), lambda i: (i,)),
        out_shape=jax.ShapeDtypeStruct((C,), jnp.float32),
        scratch_shapes=[pltpu.VMEM((N, ch_tile), jnp.int32)],
    )(x2)

    row_tile = 512 if N % 512 == 0 else N
    out2 = pl.pallas_call(
        _mask_body,
        grid=(N // row_tile,),
        in_specs=[
            pl.BlockSpec((row_tile, C), lambda i: (i, 0)),
            pl.BlockSpec((C,), lambda i: (0,)),
        ],
        out_specs=pl.BlockSpec((row_tile, C), lambda i: (i, 0)),
        out_shape=jax.ShapeDtypeStruct((N, C), jnp.float32),
    )(x2, thresh)
    return out2.reshape(shape)
